# baseline (speedup 1.0000x reference)
import jax
import jax.numpy as jnp
from jax.experimental import pallas as pl

EPS = 1e-5


def _mlp_kernel(x_ref, w0_ref, b0_ref, w1_ref, b1_ref, w2_ref, b2_ref,
                gamma_ref, beta_ref, out_ref):
    n = x_ref.shape[0]
    inv_n = jnp.float32(1.0 / n)
    gamma = gamma_ref[...]
    beta = beta_ref[...]

    h = jnp.dot(x_ref[...], w0_ref[...], preferred_element_type=jnp.float32)
    h = h + b0_ref[...]
    mean = jnp.sum(h, axis=0, keepdims=True) * inv_n
    d = h - mean
    var = jnp.sum(d * d, axis=0, keepdims=True) * inv_n
    h = d * (gamma * jax.lax.rsqrt(var + EPS)) + beta
    h = jnp.tanh(h)

    h = jnp.dot(h, w1_ref[...], preferred_element_type=jnp.float32)
    h = h + b1_ref[...]
    mean = jnp.sum(h, axis=0, keepdims=True) * inv_n
    d = h - mean
    var = jnp.sum(d * d, axis=0, keepdims=True) * inv_n
    h = d * (gamma * jax.lax.rsqrt(var + EPS)) + beta
    h = jnp.tanh(h)

    h = jnp.dot(h, w2_ref[...], preferred_element_type=jnp.float32)
    out_ref[...] = h + b2_ref[...]


def kernel(x, edge_index, edge_attr, W0, b0, W1, b1, W2, b2, gamma, beta):
    del edge_index, edge_attr
    n, _ = x.shape
    d_out = W2.shape[0]
    return pl.pallas_call(
        _mlp_kernel,
        out_shape=jax.ShapeDtypeStruct((n, d_out), jnp.float32),
    )(
        x,
        W0.T, b0[None, :],
        W1.T, b1[None, :],
        W2.T, b2[None, :],
        gamma[None, :], beta[None, :],
    )
